# baseline (speedup 1.0000x reference)
import functools

import jax
import jax.numpy as jnp
from jax import lax
from jax.experimental import pallas as pl
from jax.experimental.pallas import tpu as pltpu
from jax.experimental.pallas import tpu_sc as plsc

N = 10000
E = 320000
EPAD = 327680
RPT = 80
ACC_ROWS = 10240
ZPT = ACC_ROWS // 16


def _make_prop(D):
    mesh = plsc.VectorSubcoreMesh(core_axis_name="c", subcore_axis_name="s")
    CH = 8

    @functools.partial(
        pl.kernel, mesh=mesh,
        out_type=jax.ShapeDtypeStruct((2 * ACC_ROWS, D), jnp.float32),
        scratch_types=[
            pltpu.VMEM((CH, 128), jnp.int32),
            pltpu.VMEM((CH, 128), jnp.int32),
            pltpu.VMEM((128, D), jnp.float32),
            pltpu.VMEM((128, D), jnp.float32),
            pltpu.VMEM_SHARED((ACC_ROWS, D), jnp.float32),
            pltpu.SemaphoreType.DMA,
            pltpu.SemaphoreType.DMA,
        ],
    )
    def prop(row_hbm, col_hbm, z_hbm, zeros_hbm, out_hbm,
             row_v, col_v, b0, b1, acc, g0, g1):
        c = lax.axis_index("c")
        s = lax.axis_index("s")
        wid = c * 16 + s
        bufs = (b0, b1)
        gsems = (g0, g1)

        pltpu.sync_copy(zeros_hbm.at[pl.ds(s * ZPT, ZPT)],
                        acc.at[pl.ds(s * ZPT, ZPT)])
        plsc.subcore_barrier()

        def chunk(kk, carry):
            base = wid * RPT + kk * CH
            pltpu.sync_copy(row_hbm.at[pl.ds(base, CH)], row_v)
            pltpu.sync_copy(col_hbm.at[pl.ds(base, CH)], col_v)
            d_prev = pltpu.async_copy(z_hbm.at[row_v.at[0]], b0, g0)
            for t in range(1, CH):
                d_cur = pltpu.async_copy(z_hbm.at[row_v.at[t]],
                                         bufs[t % 2], gsems[t % 2])
                d_prev.wait()
                pltpu.sync_copy(bufs[(t - 1) % 2],
                                acc.at[col_v.at[t - 1]], add=True)
                d_prev = d_cur
            d_prev.wait()
            pltpu.sync_copy(bufs[(CH - 1) % 2],
                            acc.at[col_v.at[CH - 1]], add=True)
            return carry

        lax.fori_loop(0, RPT // CH, chunk, 0)
        plsc.subcore_barrier()
        pltpu.sync_copy(acc.at[pl.ds(s * ZPT, ZPT)],
                        out_hbm.at[pl.ds(c * ACC_ROWS + s * ZPT, ZPT)])

    return prop


_prop128 = _make_prop(128)

_B = 2000


def _rs(dA, dB):
    return lax.rsqrt(dA[:, :1] + dB[:, :1] + 1.0)


def _tc1_body(x_ref, w_ref, b_ref, dA_ref, dB_ref, z_ref):
    s = _rs(dA_ref[...], dB_ref[...])
    y = lax.dot_general(x_ref[...], w_ref[...], (((1,), (1,)), ((), ())),
                        preferred_element_type=jnp.float32)
    z_ref[...] = s * (y + b_ref[...])


def _tc2_body(aA_ref, aB_ref, z1_ref, dA_ref, dB_ref, w_ref, b_ref, z2_ref):
    s = _rs(dA_ref[...], dB_ref[...])
    h = jnp.maximum(s * (aA_ref[...] + aB_ref[...] + z1_ref[...]), 0.0)
    y = lax.dot_general(h, w_ref[...], (((1,), (1,)), ((), ())),
                        preferred_element_type=jnp.float32)
    z2_ref[...] = s * (y + b_ref[...])


def _tc3_body(aA_ref, aB_ref, z2_ref, dA_ref, dB_ref, out_ref):
    s = _rs(dA_ref[...], dB_ref[...])
    o = s * (aA_ref[:, :64] + aB_ref[:, :64] + z2_ref[:, :64])
    m = jnp.max(o, axis=1, keepdims=True)
    e = o - m
    out_ref[...] = e - jnp.log(jnp.sum(jnp.exp(e), axis=1, keepdims=True))


def _row_spec(d):
    return pl.BlockSpec((_B, d), lambda i: (i, 0))


def _full_spec(r, d):
    return pl.BlockSpec((r, d), lambda i: (0, 0))


def _tc1(x, W1, b1, dA, dB):
    return pl.pallas_call(
        _tc1_body, grid=(N // _B,),
        in_specs=[_row_spec(128), _full_spec(128, 128), _full_spec(1, 128),
                  _row_spec(128), _row_spec(128)],
        out_specs=_row_spec(128),
        out_shape=jax.ShapeDtypeStruct((N, 128), jnp.float32),
    )(x, W1, b1, dA, dB)


def _tc2(aA, aB, z1, dA, dB, W2p, b2p):
    return pl.pallas_call(
        _tc2_body, grid=(N // _B,),
        in_specs=[_row_spec(128), _row_spec(128), _row_spec(128),
                  _row_spec(128), _row_spec(128),
                  _full_spec(128, 128), _full_spec(1, 128)],
        out_specs=_row_spec(128),
        out_shape=jax.ShapeDtypeStruct((N, 128), jnp.float32),
    )(aA, aB, z1, dA, dB, W2p, b2p)


def _tc3(aA, aB, z2, dA, dB):
    return pl.pallas_call(
        _tc3_body, grid=(N // _B,),
        in_specs=[_row_spec(128), _row_spec(128), _row_spec(128),
                  _row_spec(128), _row_spec(128)],
        out_specs=_row_spec(64),
        out_shape=jax.ShapeDtypeStruct((N, 64), jnp.float32),
    )(aA, aB, z2, dA, dB)


def kernel(x, edge_index, W1, b1, W2, b2):
    ei = edge_index.astype(jnp.int32)
    row, col = ei[0], ei[1]
    pad = EPAD - E
    row_deg = jnp.concatenate([row, jnp.full((pad,), N, jnp.int32)])
    row_g = jnp.concatenate([row, jnp.zeros((pad,), jnp.int32)])
    col_s = jnp.concatenate([col, jnp.full((pad,), N, jnp.int32)])
    row_deg = row_deg.reshape(EPAD // 128, 128)
    row_g = row_g.reshape(EPAD // 128, 128)
    col_s = col_s.reshape(EPAD // 128, 128)

    ones_tab = jnp.ones((N, 128), jnp.float32)
    zeros_idx = jnp.zeros((EPAD // 128, 128), jnp.int32)
    zeros128 = jnp.zeros((ACC_ROWS, 128), jnp.float32)
    W2p = jnp.concatenate([W2, jnp.zeros((64, 128), jnp.float32)], axis=0)
    b2p = jnp.concatenate([b2, jnp.zeros((64,), jnp.float32)]).reshape(1, 128)

    deg2 = _prop128(zeros_idx, row_deg, ones_tab, zeros128)
    dA, dB = deg2[:N], deg2[ACC_ROWS:ACC_ROWS + N]

    z1 = _tc1(x, W1, b1.reshape(1, 128), dA, dB)
    agg1 = _prop128(row_g, col_s, z1, zeros128)
    z2 = _tc2(agg1[:N], agg1[ACC_ROWS:ACC_ROWS + N], z1, dA, dB, W2p, b2p)
    agg2 = _prop128(row_g, col_s, z2, zeros128)
    return _tc3(agg2[:N], agg2[ACC_ROWS:ACC_ROWS + N], z2, dA, dB)


# speedup vs baseline: 31.1421x; 31.1421x over previous
import functools

import jax
import jax.numpy as jnp
from jax import lax
from jax.experimental import pallas as pl
from jax.experimental.pallas import tpu as pltpu
from jax.experimental.pallas import tpu_sc as plsc

N = 10000
E = 320000
EPAD = 327680
RPT = 80
ACC_ROWS = 10240
ZPT = ACC_ROWS // 16


def _make_deg():
    mesh = plsc.VectorSubcoreMesh(core_axis_name="c", subcore_axis_name="s")

    @functools.partial(
        pl.kernel, mesh=mesh,
        out_type=jax.ShapeDtypeStruct((2 * ACC_ROWS, 128), jnp.float32),
        scratch_types=[
            pltpu.VMEM((RPT, 128), jnp.int32),
            pltpu.VMEM((128, 128), jnp.float32),
            pltpu.VMEM_SHARED((ACC_ROWS, 128), jnp.float32),
        ],
    )
    def deg(row_hbm, ones_hbm, zeros_hbm, out_hbm, idx_v, ones_v, acc):
        c = lax.axis_index("c")
        s = lax.axis_index("s")
        wid = c * 16 + s

        pltpu.sync_copy(zeros_hbm.at[pl.ds(s * ZPT, ZPT)],
                        acc.at[pl.ds(s * ZPT, ZPT)])
        pltpu.sync_copy(ones_hbm, ones_v)
        pltpu.sync_copy(row_hbm.at[pl.ds(wid * RPT, RPT)], idx_v)
        plsc.subcore_barrier()

        def body(j, carry):
            pltpu.sync_copy(ones_v, acc.at[idx_v.at[j]], add=True)
            return carry

        lax.fori_loop(0, RPT, body, 0)
        plsc.subcore_barrier()
        pltpu.sync_copy(acc.at[pl.ds(s * ZPT, ZPT)],
                        out_hbm.at[pl.ds(c * ACC_ROWS + s * ZPT, ZPT)])

    return deg


def _make_prop(D):
    mesh = plsc.VectorSubcoreMesh(core_axis_name="c", subcore_axis_name="s")
    CH = 8

    @functools.partial(
        pl.kernel, mesh=mesh,
        out_type=jax.ShapeDtypeStruct((2 * ACC_ROWS, D), jnp.float32),
        scratch_types=[
            pltpu.VMEM((CH, 128), jnp.int32),
            pltpu.VMEM((CH, 128), jnp.int32),
            pltpu.VMEM((128, D), jnp.float32),
            pltpu.VMEM((128, D), jnp.float32),
            pltpu.VMEM_SHARED((ACC_ROWS, D), jnp.float32),
            pltpu.SemaphoreType.DMA,
            pltpu.SemaphoreType.DMA,
        ],
    )
    def prop(row_hbm, col_hbm, z_hbm, zeros_hbm, out_hbm,
             row_v, col_v, b0, b1, acc, g0, g1):
        c = lax.axis_index("c")
        s = lax.axis_index("s")
        wid = c * 16 + s
        bufs = (b0, b1)
        gsems = (g0, g1)

        pltpu.sync_copy(zeros_hbm.at[pl.ds(s * ZPT, ZPT)],
                        acc.at[pl.ds(s * ZPT, ZPT)])
        plsc.subcore_barrier()

        def chunk(kk, carry):
            base = wid * RPT + kk * CH
            pltpu.sync_copy(row_hbm.at[pl.ds(base, CH)], row_v)
            pltpu.sync_copy(col_hbm.at[pl.ds(base, CH)], col_v)
            d_prev = pltpu.async_copy(z_hbm.at[row_v.at[0]], b0, g0)
            for t in range(1, CH):
                d_cur = pltpu.async_copy(z_hbm.at[row_v.at[t]],
                                         bufs[t % 2], gsems[t % 2])
                d_prev.wait()
                pltpu.sync_copy(bufs[(t - 1) % 2],
                                acc.at[col_v.at[t - 1]], add=True)
                d_prev = d_cur
            d_prev.wait()
            pltpu.sync_copy(bufs[(CH - 1) % 2],
                            acc.at[col_v.at[CH - 1]], add=True)
            return carry

        lax.fori_loop(0, RPT // CH, chunk, 0)
        plsc.subcore_barrier()
        pltpu.sync_copy(acc.at[pl.ds(s * ZPT, ZPT)],
                        out_hbm.at[pl.ds(c * ACC_ROWS + s * ZPT, ZPT)])

    return prop


_deg = _make_deg()
_prop128 = _make_prop(128)

_B = 2000


def _rs(dA, dB):
    return lax.rsqrt(dA[:, :1] + dB[:, :1] + 1.0)


def _tc1_body(x_ref, w_ref, b_ref, dA_ref, dB_ref, z_ref):
    s = _rs(dA_ref[...], dB_ref[...])
    y = lax.dot_general(x_ref[...], w_ref[...], (((1,), (1,)), ((), ())),
                        preferred_element_type=jnp.float32)
    z_ref[...] = s * (y + b_ref[...])


def _tc2_body(aA_ref, aB_ref, z1_ref, dA_ref, dB_ref, w_ref, b_ref, z2_ref):
    s = _rs(dA_ref[...], dB_ref[...])
    h = jnp.maximum(s * (aA_ref[...] + aB_ref[...] + z1_ref[...]), 0.0)
    y = lax.dot_general(h, w_ref[...], (((1,), (1,)), ((), ())),
                        preferred_element_type=jnp.float32)
    z2_ref[...] = s * (y + b_ref[...])


def _tc3_body(aA_ref, aB_ref, z2_ref, dA_ref, dB_ref, out_ref):
    s = _rs(dA_ref[...], dB_ref[...])
    o = s * (aA_ref[:, :64] + aB_ref[:, :64] + z2_ref[:, :64])
    m = jnp.max(o, axis=1, keepdims=True)
    e = o - m
    out_ref[...] = e - jnp.log(jnp.sum(jnp.exp(e), axis=1, keepdims=True))


def _row_spec(d):
    return pl.BlockSpec((_B, d), lambda i: (i, 0))


def _full_spec(r, d):
    return pl.BlockSpec((r, d), lambda i: (0, 0))


def _tc1(x, W1, b1, dA, dB):
    return pl.pallas_call(
        _tc1_body, grid=(N // _B,),
        in_specs=[_row_spec(128), _full_spec(128, 128), _full_spec(1, 128),
                  _row_spec(128), _row_spec(128)],
        out_specs=_row_spec(128),
        out_shape=jax.ShapeDtypeStruct((N, 128), jnp.float32),
    )(x, W1, b1, dA, dB)


def _tc2(aA, aB, z1, dA, dB, W2p, b2p):
    return pl.pallas_call(
        _tc2_body, grid=(N // _B,),
        in_specs=[_row_spec(128), _row_spec(128), _row_spec(128),
                  _row_spec(128), _row_spec(128),
                  _full_spec(128, 128), _full_spec(1, 128)],
        out_specs=_row_spec(128),
        out_shape=jax.ShapeDtypeStruct((N, 128), jnp.float32),
    )(aA, aB, z1, dA, dB, W2p, b2p)


def _tc3(aA, aB, z2, dA, dB):
    return pl.pallas_call(
        _tc3_body, grid=(N // _B,),
        in_specs=[_row_spec(128), _row_spec(128), _row_spec(128),
                  _row_spec(128), _row_spec(128)],
        out_specs=_row_spec(64),
        out_shape=jax.ShapeDtypeStruct((N, 64), jnp.float32),
    )(aA, aB, z2, dA, dB)


def kernel(x, edge_index, W1, b1, W2, b2):
    ei = edge_index.astype(jnp.int32)
    row, col = ei[0], ei[1]
    pad = EPAD - E
    row_deg = jnp.concatenate([row, jnp.full((pad,), N, jnp.int32)])
    row_g = jnp.concatenate([row, jnp.arange(pad, dtype=jnp.int32) % N])
    col_s = jnp.concatenate([col, jnp.full((pad,), N, jnp.int32)])
    row_deg = row_deg.reshape(EPAD // 128, 128)
    row_g = row_g.reshape(EPAD // 128, 128)
    col_s = col_s.reshape(EPAD // 128, 128)

    ones128 = jnp.ones((128, 128), jnp.float32)
    zeros128 = jnp.zeros((ACC_ROWS, 128), jnp.float32)
    W2p = jnp.concatenate([W2, jnp.zeros((64, 128), jnp.float32)], axis=0)
    b2p = jnp.concatenate([b2, jnp.zeros((64,), jnp.float32)]).reshape(1, 128)

    deg2 = _deg(row_deg, ones128, zeros128)
    dA, dB = deg2[:N], deg2[ACC_ROWS:ACC_ROWS + N]

    z1 = _tc1(x, W1, b1.reshape(1, 128), dA, dB)
    agg1 = _prop128(row_g, col_s, z1, zeros128)
    z2 = _tc2(agg1[:N], agg1[ACC_ROWS:ACC_ROWS + N], z1, dA, dB, W2p, b2p)
    agg2 = _prop128(row_g, col_s, z2, zeros128)
    return _tc3(agg2[:N], agg2[ACC_ROWS:ACC_ROWS + N], z2, dA, dB)


# speedup vs baseline: 31.5617x; 1.0135x over previous
import functools

import jax
import jax.numpy as jnp
from jax import lax
from jax.experimental import pallas as pl
from jax.experimental.pallas import tpu as pltpu
from jax.experimental.pallas import tpu_sc as plsc

N = 10000
E = 320000
EPAD = 327680
RPT = 80
ACC_ROWS = 10240
ZPT = ACC_ROWS // 16


def _make_deg():
    mesh = plsc.VectorSubcoreMesh(core_axis_name="c", subcore_axis_name="s")

    @functools.partial(
        pl.kernel, mesh=mesh,
        out_type=jax.ShapeDtypeStruct((2 * ACC_ROWS, 128), jnp.float32),
        scratch_types=[
            pltpu.VMEM((RPT, 128), jnp.int32),
            pltpu.VMEM((128, 128), jnp.float32),
            pltpu.VMEM_SHARED((ACC_ROWS, 128), jnp.float32),
        ],
    )
    def deg(row_hbm, ones_hbm, zeros_hbm, out_hbm, idx_v, ones_v, acc):
        c = lax.axis_index("c")
        s = lax.axis_index("s")
        wid = c * 16 + s

        pltpu.sync_copy(zeros_hbm.at[pl.ds(s * ZPT, ZPT)],
                        acc.at[pl.ds(s * ZPT, ZPT)])
        pltpu.sync_copy(ones_hbm, ones_v)
        pltpu.sync_copy(row_hbm.at[pl.ds(wid * RPT, RPT)], idx_v)
        plsc.subcore_barrier()

        def body(j, carry):
            pltpu.sync_copy(ones_v, acc.at[idx_v.at[j]], add=True)
            return carry

        lax.fori_loop(0, RPT, body, 0)
        plsc.subcore_barrier()
        pltpu.sync_copy(acc.at[pl.ds(s * ZPT, ZPT)],
                        out_hbm.at[pl.ds(c * ACC_ROWS + s * ZPT, ZPT)])

    return deg


def _make_prop(D):
    mesh = plsc.VectorSubcoreMesh(core_axis_name="c", subcore_axis_name="s")
    CH = 8

    @functools.partial(
        pl.kernel, mesh=mesh,
        out_type=jax.ShapeDtypeStruct((2 * ACC_ROWS, D), jnp.float32),
        scratch_types=[
            pltpu.VMEM((CH, 128), jnp.int32),
            pltpu.VMEM((CH, 128), jnp.int32),
            pltpu.VMEM((128, D), jnp.float32),
            pltpu.VMEM((128, D), jnp.float32),
            pltpu.VMEM_SHARED((ACC_ROWS, D), jnp.float32),
            pltpu.SemaphoreType.DMA,
            pltpu.SemaphoreType.DMA,
        ],
    )
    def prop(row_hbm, col_hbm, z_hbm, zeros_hbm, out_hbm,
             row_v, col_v, b0, b1, acc, g0, g1):
        c = lax.axis_index("c")
        s = lax.axis_index("s")
        wid = c * 16 + s
        bufs = (b0, b1)
        gsems = (g0, g1)

        pltpu.sync_copy(zeros_hbm.at[pl.ds(s * ZPT, ZPT)],
                        acc.at[pl.ds(s * ZPT, ZPT)])
        plsc.subcore_barrier()

        def chunk(kk, carry):
            base = wid * RPT + kk * CH
            pltpu.sync_copy(row_hbm.at[pl.ds(base, CH)], row_v)
            pltpu.sync_copy(col_hbm.at[pl.ds(base, CH)], col_v)
            d_prev = pltpu.async_copy(z_hbm.at[row_v.at[0]], b0, g0)
            for t in range(1, CH):
                d_cur = pltpu.async_copy(z_hbm.at[row_v.at[t]],
                                         bufs[t % 2], gsems[t % 2])
                d_prev.wait()
                pltpu.sync_copy(bufs[(t - 1) % 2],
                                acc.at[col_v.at[t - 1]], add=True)
                d_prev = d_cur
            d_prev.wait()
            pltpu.sync_copy(bufs[(CH - 1) % 2],
                            acc.at[col_v.at[CH - 1]], add=True)
            return carry

        lax.fori_loop(0, RPT // CH, chunk, 0)
        plsc.subcore_barrier()
        pltpu.sync_copy(acc.at[pl.ds(s * ZPT, ZPT)],
                        out_hbm.at[pl.ds(c * ACC_ROWS + s * ZPT, ZPT)])

    return prop


_deg = _make_deg()
_prop128 = _make_prop(128)

_B = 640
_G = ACC_ROWS // _B


def _rs(dA, dB):
    return lax.rsqrt(dA[:, :1] + dB[:, :1] + 1.0)


def _tc1_body(x_ref, w_ref, b_ref, dA_ref, dB_ref, z_ref):
    s = _rs(dA_ref[...], dB_ref[...])
    y = lax.dot_general(x_ref[...], w_ref[...], (((1,), (1,)), ((), ())),
                        preferred_element_type=jnp.float32)
    z_ref[...] = s * (y + b_ref[...])


def _tc2_body(agA_ref, agB_ref, z1_ref, dA_ref, dB_ref, w_ref, b_ref, z2_ref):
    s = _rs(dA_ref[...], dB_ref[...])
    h = jnp.maximum(s * (agA_ref[...] + agB_ref[...] + z1_ref[...]), 0.0)
    y = lax.dot_general(h, w_ref[...], (((1,), (1,)), ((), ())),
                        preferred_element_type=jnp.float32)
    z2_ref[...] = s * (y + b_ref[...])


def _tc3_body(agA_ref, agB_ref, z2_ref, dA_ref, dB_ref, out_ref):
    s = _rs(dA_ref[...], dB_ref[...])
    o = s * (agA_ref[:, :64] + agB_ref[:, :64] + z2_ref[:, :64])
    m = jnp.max(o, axis=1, keepdims=True)
    e = o - m
    out_ref[...] = e - jnp.log(jnp.sum(jnp.exp(e), axis=1, keepdims=True))


def _row_spec(d):
    return pl.BlockSpec((_B, d), lambda i: (i, 0))


def _half_specs(d):
    return (pl.BlockSpec((_B, d), lambda i: (i, 0)),
            pl.BlockSpec((_B, d), lambda i: (i + _G, 0)))


def _full_spec(r, d):
    return pl.BlockSpec((r, d), lambda i: (0, 0))


def _tc1(x, W1, b1, deg2):
    dA, dB = _half_specs(128)
    return pl.pallas_call(
        _tc1_body, grid=(_G,),
        in_specs=[_row_spec(128), _full_spec(128, 128), _full_spec(1, 128),
                  dA, dB],
        out_specs=_row_spec(128),
        out_shape=jax.ShapeDtypeStruct((ACC_ROWS, 128), jnp.float32),
    )(x, W1, b1, deg2, deg2)


def _tc2(agg1, z1, deg2, W2p, b2p):
    aA, aB = _half_specs(128)
    dA, dB = _half_specs(128)
    return pl.pallas_call(
        _tc2_body, grid=(_G,),
        in_specs=[aA, aB, _row_spec(128), dA, dB,
                  _full_spec(128, 128), _full_spec(1, 128)],
        out_specs=_row_spec(128),
        out_shape=jax.ShapeDtypeStruct((ACC_ROWS, 128), jnp.float32),
    )(agg1, agg1, z1, deg2, deg2, W2p, b2p)


def _tc3(agg2, z2, deg2):
    aA, aB = _half_specs(128)
    dA, dB = _half_specs(128)
    return pl.pallas_call(
        _tc3_body, grid=(_G,),
        in_specs=[aA, aB, _row_spec(128), dA, dB],
        out_specs=pl.BlockSpec((_B, 64), lambda i: (i, 0)),
        out_shape=jax.ShapeDtypeStruct((N, 64), jnp.float32),
    )(agg2, agg2, z2, deg2, deg2)


def kernel(x, edge_index, W1, b1, W2, b2):
    ei = edge_index.astype(jnp.int32)
    row, col = ei[0], ei[1]
    pad = EPAD - E
    row_deg = jnp.concatenate([row, jnp.full((pad,), N, jnp.int32)])
    row_g = jnp.concatenate([row, jnp.arange(pad, dtype=jnp.int32) % N])
    col_s = jnp.concatenate([col, jnp.full((pad,), N, jnp.int32)])
    row_deg = row_deg.reshape(EPAD // 128, 128)
    row_g = row_g.reshape(EPAD // 128, 128)
    col_s = col_s.reshape(EPAD // 128, 128)

    ones128 = jnp.ones((128, 128), jnp.float32)
    zeros128 = jnp.zeros((ACC_ROWS, 128), jnp.float32)
    W2p = jnp.concatenate([W2, jnp.zeros((64, 128), jnp.float32)], axis=0)
    b2p = jnp.concatenate([b2, jnp.zeros((64,), jnp.float32)]).reshape(1, 128)

    deg2 = _deg(row_deg, ones128, zeros128)
    z1 = _tc1(x, W1, b1.reshape(1, 128), deg2)
    agg1 = _prop128(row_g, col_s, z1, zeros128)
    z2 = _tc2(agg1, z1, deg2, W2p, b2p)
    agg2 = _prop128(row_g, col_s, z2, zeros128)
    return _tc3(agg2, z2, deg2)


# speedup vs baseline: 31.5842x; 1.0007x over previous
import functools

import jax
import jax.numpy as jnp
from jax import lax
from jax.experimental import pallas as pl
from jax.experimental.pallas import tpu as pltpu
from jax.experimental.pallas import tpu_sc as plsc

N = 10000
E = 320000
EPAD = 327680
RPT = 80
ACC_ROWS = 10240
ZPT = ACC_ROWS // 16


def _make_deg():
    mesh = plsc.VectorSubcoreMesh(core_axis_name="c", subcore_axis_name="s")

    @functools.partial(
        pl.kernel, mesh=mesh,
        out_type=jax.ShapeDtypeStruct((2 * ACC_ROWS, 128), jnp.float32),
        scratch_types=[
            pltpu.VMEM((RPT, 128), jnp.int32),
            pltpu.VMEM((128, 128), jnp.float32),
            pltpu.VMEM_SHARED((ACC_ROWS, 128), jnp.float32),
        ],
    )
    def deg(row_hbm, ones_hbm, zeros_hbm, out_hbm, idx_v, ones_v, acc):
        c = lax.axis_index("c")
        s = lax.axis_index("s")
        wid = c * 16 + s

        pltpu.sync_copy(zeros_hbm.at[pl.ds(s * ZPT, ZPT)],
                        acc.at[pl.ds(s * ZPT, ZPT)])
        pltpu.sync_copy(ones_hbm, ones_v)
        pltpu.sync_copy(row_hbm.at[pl.ds(wid * RPT, RPT)], idx_v)
        plsc.subcore_barrier()

        def body(j, carry):
            pltpu.sync_copy(ones_v, acc.at[idx_v.at[j]], add=True)
            return carry

        lax.fori_loop(0, RPT, body, 0)
        plsc.subcore_barrier()
        pltpu.sync_copy(acc.at[pl.ds(s * ZPT, ZPT)],
                        out_hbm.at[pl.ds(c * ACC_ROWS + s * ZPT, ZPT)])

    return deg


def _make_prop(D):
    mesh = plsc.VectorSubcoreMesh(core_axis_name="c", subcore_axis_name="s")
    CH = 8

    @functools.partial(
        pl.kernel, mesh=mesh,
        out_type=jax.ShapeDtypeStruct((2 * ACC_ROWS, D), jnp.float32),
        scratch_types=[
            pltpu.VMEM((CH, 128), jnp.int32),
            pltpu.VMEM((CH, 128), jnp.int32),
            pltpu.VMEM((128, D), jnp.float32),
            pltpu.VMEM((128, D), jnp.float32),
            pltpu.VMEM_SHARED((ACC_ROWS, D), jnp.float32),
            pltpu.SemaphoreType.DMA,
            pltpu.SemaphoreType.DMA,
            pltpu.SemaphoreType.DMA,
            pltpu.SemaphoreType.DMA,
        ],
    )
    def prop(row_hbm, col_hbm, z_hbm, zeros_hbm, out_hbm,
             row_v, col_v, b0, b1, acc, g0, g1, ss0, ss1):
        c = lax.axis_index("c")
        s = lax.axis_index("s")
        wid = c * 16 + s
        bufs = (b0, b1)
        gsems = (g0, g1)
        ssems = (ss0, ss1)

        pltpu.sync_copy(zeros_hbm.at[pl.ds(s * ZPT, ZPT)],
                        acc.at[pl.ds(s * ZPT, ZPT)])
        plsc.subcore_barrier()

        def chunk(kk, carry):
            base = wid * RPT + kk * CH
            pltpu.sync_copy(row_hbm.at[pl.ds(base, CH)], row_v)
            pltpu.sync_copy(col_hbm.at[pl.ds(base, CH)], col_v)
            sc = [None] * CH
            d_prev = pltpu.async_copy(z_hbm.at[row_v.at[0]], b0, g0)
            for t in range(1, CH):
                b = t % 2
                if t >= 2:
                    sc[t - 2].wait()
                d_cur = pltpu.async_copy(z_hbm.at[row_v.at[t]],
                                         bufs[b], gsems[b])
                d_prev.wait()
                sc[t - 1] = pltpu.async_copy(
                    bufs[1 - b], acc.at[col_v.at[t - 1]], ssems[1 - b],
                    add=True)
                d_prev = d_cur
            d_prev.wait()
            sc[CH - 1] = pltpu.async_copy(
                bufs[(CH - 1) % 2], acc.at[col_v.at[CH - 1]],
                ssems[(CH - 1) % 2], add=True)
            sc[CH - 2].wait()
            sc[CH - 1].wait()
            return carry

        lax.fori_loop(0, RPT // CH, chunk, 0)
        plsc.subcore_barrier()
        pltpu.sync_copy(acc.at[pl.ds(s * ZPT, ZPT)],
                        out_hbm.at[pl.ds(c * ACC_ROWS + s * ZPT, ZPT)])

    return prop


_deg = _make_deg()
_prop128 = _make_prop(128)

_B = 640
_G = ACC_ROWS // _B


def _rs(dA, dB):
    return lax.rsqrt(dA[:, :1] + dB[:, :1] + 1.0)


def _tc1_body(x_ref, w_ref, b_ref, dA_ref, dB_ref, z_ref):
    s = _rs(dA_ref[...], dB_ref[...])
    y = lax.dot_general(x_ref[...], w_ref[...], (((1,), (1,)), ((), ())),
                        preferred_element_type=jnp.float32)
    z_ref[...] = s * (y + b_ref[...])


def _tc2_body(agA_ref, agB_ref, z1_ref, dA_ref, dB_ref, w_ref, b_ref, z2_ref):
    s = _rs(dA_ref[...], dB_ref[...])
    h = jnp.maximum(s * (agA_ref[...] + agB_ref[...] + z1_ref[...]), 0.0)
    y = lax.dot_general(h, w_ref[...], (((1,), (1,)), ((), ())),
                        preferred_element_type=jnp.float32)
    z2_ref[...] = s * (y + b_ref[...])


def _tc3_body(agA_ref, agB_ref, z2_ref, dA_ref, dB_ref, out_ref):
    s = _rs(dA_ref[...], dB_ref[...])
    o = s * (agA_ref[:, :64] + agB_ref[:, :64] + z2_ref[:, :64])
    m = jnp.max(o, axis=1, keepdims=True)
    e = o - m
    out_ref[...] = e - jnp.log(jnp.sum(jnp.exp(e), axis=1, keepdims=True))


def _row_spec(d):
    return pl.BlockSpec((_B, d), lambda i: (i, 0))


def _half_specs(d):
    return (pl.BlockSpec((_B, d), lambda i: (i, 0)),
            pl.BlockSpec((_B, d), lambda i: (i + _G, 0)))


def _full_spec(r, d):
    return pl.BlockSpec((r, d), lambda i: (0, 0))


def _tc1(x, W1, b1, deg2):
    dA, dB = _half_specs(128)
    return pl.pallas_call(
        _tc1_body, grid=(_G,),
        in_specs=[_row_spec(128), _full_spec(128, 128), _full_spec(1, 128),
                  dA, dB],
        out_specs=_row_spec(128),
        out_shape=jax.ShapeDtypeStruct((ACC_ROWS, 128), jnp.float32),
    )(x, W1, b1, deg2, deg2)


def _tc2(agg1, z1, deg2, W2p, b2p):
    aA, aB = _half_specs(128)
    dA, dB = _half_specs(128)
    return pl.pallas_call(
        _tc2_body, grid=(_G,),
        in_specs=[aA, aB, _row_spec(128), dA, dB,
                  _full_spec(128, 128), _full_spec(1, 128)],
        out_specs=_row_spec(128),
        out_shape=jax.ShapeDtypeStruct((ACC_ROWS, 128), jnp.float32),
    )(agg1, agg1, z1, deg2, deg2, W2p, b2p)


def _tc3(agg2, z2, deg2):
    aA, aB = _half_specs(128)
    dA, dB = _half_specs(128)
    return pl.pallas_call(
        _tc3_body, grid=(_G,),
        in_specs=[aA, aB, _row_spec(128), dA, dB],
        out_specs=pl.BlockSpec((_B, 64), lambda i: (i, 0)),
        out_shape=jax.ShapeDtypeStruct((N, 64), jnp.float32),
    )(agg2, agg2, z2, deg2, deg2)


def kernel(x, edge_index, W1, b1, W2, b2):
    ei = edge_index.astype(jnp.int32)
    row, col = ei[0], ei[1]
    pad = EPAD - E
    row_deg = jnp.concatenate([row, jnp.full((pad,), N, jnp.int32)])
    row_g = jnp.concatenate([row, jnp.arange(pad, dtype=jnp.int32) % N])
    col_s = jnp.concatenate([col, jnp.full((pad,), N, jnp.int32)])
    row_deg = row_deg.reshape(EPAD // 128, 128)
    row_g = row_g.reshape(EPAD // 128, 128)
    col_s = col_s.reshape(EPAD // 128, 128)

    ones128 = jnp.ones((128, 128), jnp.float32)
    zeros128 = jnp.zeros((ACC_ROWS, 128), jnp.float32)
    W2p = jnp.concatenate([W2, jnp.zeros((64, 128), jnp.float32)], axis=0)
    b2p = jnp.concatenate([b2, jnp.zeros((64,), jnp.float32)]).reshape(1, 128)

    deg2 = _deg(row_deg, ones128, zeros128)
    z1 = _tc1(x, W1, b1.reshape(1, 128), deg2)
    agg1 = _prop128(row_g, col_s, z1, zeros128)
    z2 = _tc2(agg1, z1, deg2, W2p, b2p)
    agg2 = _prop128(row_g, col_s, z2, zeros128)
    return _tc3(agg2, z2, deg2)
